# baseline (speedup 1.0000x reference)
import functools

import jax
import jax.numpy as jnp
import numpy as np
from jax import lax
from jax.experimental import pallas as pl
from jax.experimental.pallas import tpu as pltpu
from jax.experimental.pallas import tpu_sc as plsc

_B = 16384
_NSP = 26
_EMB = 32
_VOCAB = 100000
_NF = 27
_ROWS = _B * _NSP

_NC = 2
_NSUB = 16
_NW = _NC * _NSUB
_RPW = _ROWS // _NW
_CH = 128
_NCH = _RPW // _CH

_BS = 512
_NBLK = _B // _BS


def _sc_gather_body(tables_h, idx_h, out_h,
                    idx_v, qa, qb, pa, pb, oa, ob, sga, sgb, swa, swb):
    c = lax.axis_index("c")
    s = lax.axis_index("s")
    wid = s * _NC + c
    pltpu.sync_copy(idx_h.at[wid], idx_v)
    base = wid * _RPW

    def compute_q(j, q):
        for k in range(_CH // 16):
            q[pl.ds(k * 16, 16)] = lax.shift_right_logical(
                idx_v[j, pl.ds(k * 16, 16)], 2)

    def fire(q, p, sem):
        return pltpu.async_copy(tables_h.at[q], p, sem)

    def extract(j, p, o):
        for k in range(_CH // 16):
            ivec = idx_v[j, pl.ds(k * 16, 16)]
            sub = lax.shift_left(ivec & 3, 5)
            rvec = lax.iota(jnp.int32, 16) + (k * 16)
            for d in range(_EMB):
                vals = plsc.load_gather(p, [rvec, sub + d])
                plsc.store_scatter(
                    o, [rvec, jnp.full((16,), d, jnp.int32)], vals)

    def wb(j, o, wsem):
        pltpu.async_copy(o, out_h.at[pl.ds(base + j * _CH, _CH)], wsem)

    def drain_wb(o, wsem):
        pltpu.make_async_copy(o, out_h.at[pl.ds(base, _CH)], wsem).wait()

    compute_q(0, qa)
    fire(qa, pa, sga)

    def body(t, carry):
        j0 = 2 * t
        j1 = j0 + 1
        compute_q(j1, qb)
        fire(qb, pb, sgb)
        pltpu.make_async_copy(tables_h.at[qa], pa, sga).wait()

        @pl.when(t > 0)
        def _():
            drain_wb(oa, swa)
        extract(j0, pa, oa)
        wb(j0, oa, swa)

        @pl.when(t < _NCH // 2 - 1)
        def _():
            compute_q(j0 + 2, qa)
            fire(qa, pa, sga)
        pltpu.make_async_copy(tables_h.at[qb], pb, sgb).wait()

        @pl.when(t > 0)
        def _():
            drain_wb(ob, swb)
        extract(j1, pb, ob)
        wb(j1, ob, swb)
        return carry

    lax.fori_loop(0, _NCH // 2, body, 0)
    drain_wb(oa, swa)
    drain_wb(ob, swb)


@jax.jit
def _sc_gather(tables_packed, idx):
    mesh = plsc.VectorSubcoreMesh(
        core_axis_name="c", subcore_axis_name="s",
        num_cores=_NC, num_subcores=_NSUB)
    return pl.kernel(
        _sc_gather_body,
        out_type=jax.ShapeDtypeStruct((_ROWS, _EMB), jnp.float32),
        mesh=mesh,
        scratch_types=[
            pltpu.VMEM((_NCH, _CH), jnp.int32),
            pltpu.VMEM((_CH,), jnp.int32),
            pltpu.VMEM((_CH,), jnp.int32),
            pltpu.VMEM((_CH, 128), jnp.float32),
            pltpu.VMEM((_CH, 128), jnp.float32),
            pltpu.VMEM((_CH, _EMB), jnp.float32),
            pltpu.VMEM((_CH, _EMB), jnp.float32),
            pltpu.SemaphoreType.DMA,
            pltpu.SemaphoreType.DMA,
            pltpu.SemaphoreType.DMA,
            pltpu.SemaphoreType.DMA,
        ],
        compiler_params=pltpu.CompilerParams(needs_layout_passes=False),
    )(tables_packed, idx)


def _tc_body(dense_r, embs_r, bw0_r, bb0_r, bw1_r, bb1_r, bw2_r, bb2_r,
             tw0z_r, w729_r, tb0_r, tw1_r, tb1_r, tw2_r, tb2_r, out_r):
    f32 = jnp.float32
    bf = jnp.bfloat16
    h = dense_r[...].astype(bf)
    h = jnp.dot(h, bw0_r[...], preferred_element_type=f32) + bb0_r[...][None, :]
    h = jnp.maximum(h, 0).astype(bf)
    h = jnp.dot(h, bw1_r[...], preferred_element_type=f32) + bb1_r[...][None, :]
    h = jnp.maximum(h, 0).astype(bf)
    h = jnp.dot(h, bw2_r[...], preferred_element_type=f32) + bb2_r[...][None, :]
    z0 = jnp.maximum(h, 0)
    z0b = z0.astype(bf)
    embs = embs_r[...].astype(bf)
    t3 = jnp.concatenate([z0b[:, None, :], embs], axis=1)
    z = lax.dot_general(t3, t3, (((2,), (2,)), ((0,), (0,))),
                        preferred_element_type=f32)
    zf = z.reshape(_BS, _NF * _NF).astype(bf)
    y = jnp.dot(z0b, tw0z_r[...], preferred_element_type=f32)
    y = y + jnp.dot(zf, w729_r[...], preferred_element_type=f32) + tb0_r[...][None, :]
    h = jnp.maximum(y, 0).astype(bf)
    h = jnp.dot(h, tw1_r[...], preferred_element_type=f32) + tb1_r[...][None, :]
    h = jnp.maximum(h, 0).astype(bf)
    o = jnp.dot(h, tw2_r[...], preferred_element_type=f32) + tb2_r[...][None, :]
    out_r[...] = jax.nn.sigmoid(o)


def _full(shape):
    return pl.BlockSpec(shape, lambda i: tuple(0 for _ in shape))


@jax.jit
def _tc_dense(dense, embs, bw0, bb0, bw1, bb1, bw2, bb2,
              tw0z, w729, tb0, tw1, tb1, tw2, tb2):
    return pl.pallas_call(
        _tc_body,
        grid=(_NBLK,),
        in_specs=[
            pl.BlockSpec((_BS, 13), lambda i: (i, 0)),
            pl.BlockSpec((_BS, _NSP, _EMB), lambda i: (i, 0, 0)),
            _full((13, 512)), _full((512,)),
            _full((512, 256)), _full((256,)),
            _full((256, 32)), _full((32,)),
            _full((32, 512)), _full((_NF * _NF, 512)), _full((512,)),
            _full((512, 256)), _full((256,)),
            _full((256, 1)), _full((1,)),
        ],
        out_specs=pl.BlockSpec((_BS, 1), lambda i: (i, 0)),
        out_shape=jax.ShapeDtypeStruct((_B, 1), jnp.float32),
    )(dense, embs, bw0, bb0, bw1, bb1, bw2, bb2,
      tw0z, w729, tb0, tw1, tb1, tw2, tb2)


_IU0, _IU1 = np.triu_indices(_NF, k=1)
_W729_ROWS = np.asarray(_IU0 * _NF + _IU1, dtype=np.int32)


def kernel(dense, sparse, tables, bw0, bb0, bw1, bb1, bw2, bb2,
           tw0, tb0, tw1, tb1, tw2, tb2):
    idx = (sparse.astype(jnp.int32)
           + (jnp.arange(_NSP, dtype=jnp.int32) * _VOCAB)[None, :])
    idx = idx.reshape(_NW, _NCH, _CH)
    tables_packed = tables.reshape(_NSP * _VOCAB // 4, _EMB * 4)

    embs_flat = _sc_gather(tables_packed, idx)
    embs = embs_flat.reshape(_B, _NSP, _EMB)

    bf = jnp.bfloat16
    tw0z = tw0[:_EMB].astype(bf)
    w729 = jnp.zeros((_NF * _NF, 512), dtype=bf)
    w729 = w729.at[_W729_ROWS].set(tw0[_EMB:].astype(bf))

    out = _tc_dense(dense, embs,
                    bw0.astype(bf), bb0, bw1.astype(bf), bb1,
                    bw2.astype(bf), bb2,
                    tw0z, w729, tb0,
                    tw1.astype(bf), tb1, tw2.astype(bf), tb2)
    return out.reshape(_B)


# speedup vs baseline: 1.4160x; 1.4160x over previous
import functools

import jax
import jax.numpy as jnp
import numpy as np
from jax import lax
from jax.experimental import pallas as pl
from jax.experimental.pallas import tpu as pltpu
from jax.experimental.pallas import tpu_sc as plsc

_B = 16384
_NSP = 26
_EMB = 32
_VOCAB = 100000
_NF = 27
_ROWS = _B * _NSP

_NC = 2
_NSUB = 16
_NW = _NC * _NSUB
_RPW = _ROWS // _NW
_CH = 128
_NCH = _RPW // _CH

_BS = 512
_NBLK = _B // _BS


_SCCH = 4
_SCH = _CH * _SCCH
_NSCH = _RPW // _SCH


def _sc_gather_body(tables_h, idx_h, out_h, idx_v, ra, rb, sga, sgb):
    c = lax.axis_index("c")
    s = lax.axis_index("s")
    wid = s * _NC + c
    pltpu.sync_copy(idx_h.at[wid], idx_v)
    base = wid * _RPW

    def fireblock(jj, buf, sem):
        for k in range(_SCCH):
            pltpu.async_copy(tables_h.at[idx_v.at[jj * _SCCH + k]],
                             buf.at[pl.ds(k * _CH, _CH)], sem)

    def drainblock(jj, buf, sem):
        for k in range(_SCCH):
            pltpu.make_async_copy(tables_h.at[idx_v.at[jj * _SCCH + k]],
                                  buf.at[pl.ds(k * _CH, _CH)], sem).wait()

    def wb(jj, buf):
        pltpu.sync_copy(buf, out_h.at[pl.ds(base + jj * _SCH, _SCH)])

    fireblock(0, ra, sga)

    def body(t, carry):
        j0 = 2 * t
        j1 = j0 + 1
        fireblock(j1, rb, sgb)
        drainblock(j0, ra, sga)
        wb(j0, ra)

        @pl.when(t < _NSCH // 2 - 1)
        def _():
            fireblock(j0 + 2, ra, sga)
        drainblock(j1, rb, sgb)
        wb(j1, rb)
        return carry

    lax.fori_loop(0, _NSCH // 2, body, 0)


@jax.jit
def _sc_gather(tables_flat, idx):
    mesh = plsc.VectorSubcoreMesh(
        core_axis_name="c", subcore_axis_name="s",
        num_cores=_NC, num_subcores=_NSUB)
    return pl.kernel(
        _sc_gather_body,
        out_type=jax.ShapeDtypeStruct((_ROWS, _EMB), jnp.float32),
        mesh=mesh,
        scratch_types=[
            pltpu.VMEM((_NCH, _CH), jnp.int32),
            pltpu.VMEM((_SCH, _EMB), jnp.float32),
            pltpu.VMEM((_SCH, _EMB), jnp.float32),
            pltpu.SemaphoreType.DMA,
            pltpu.SemaphoreType.DMA,
        ],
        compiler_params=pltpu.CompilerParams(use_tc_tiling_on_sc=False),
    )(tables_flat, idx)


def _tc_body(dense_r, embs_r, bw0_r, bb0_r, bw1_r, bb1_r, bw2_r, bb2_r,
             tw0z_r, w729_r, tb0_r, tw1_r, tb1_r, tw2_r, tb2_r, out_r):
    f32 = jnp.float32
    bf = jnp.bfloat16
    h = dense_r[...].astype(bf)
    h = jnp.dot(h, bw0_r[...], preferred_element_type=f32) + bb0_r[...][None, :]
    h = jnp.maximum(h, 0).astype(bf)
    h = jnp.dot(h, bw1_r[...], preferred_element_type=f32) + bb1_r[...][None, :]
    h = jnp.maximum(h, 0).astype(bf)
    h = jnp.dot(h, bw2_r[...], preferred_element_type=f32) + bb2_r[...][None, :]
    z0 = jnp.maximum(h, 0)
    z0b = z0.astype(bf)
    embs = embs_r[...].astype(bf)
    t2 = jnp.concatenate([z0b, embs], axis=1)
    t3 = t2.reshape(_BS, _NF, _EMB)
    z = lax.dot_general(t3, t3, (((2,), (2,)), ((0,), (0,))),
                        preferred_element_type=f32)
    zf = z.reshape(_BS, _NF * _NF).astype(bf)
    y = jnp.dot(z0b, tw0z_r[...], preferred_element_type=f32)
    y = y + jnp.dot(zf, w729_r[...], preferred_element_type=f32) + tb0_r[...][None, :]
    h = jnp.maximum(y, 0).astype(bf)
    h = jnp.dot(h, tw1_r[...], preferred_element_type=f32) + tb1_r[...][None, :]
    h = jnp.maximum(h, 0).astype(bf)
    o = jnp.dot(h, tw2_r[...], preferred_element_type=f32) + tb2_r[...][None, :]
    out_r[...] = jax.nn.sigmoid(o)


def _full(shape):
    return pl.BlockSpec(shape, lambda i: tuple(0 for _ in shape))


@jax.jit
def _tc_dense(dense, embs, bw0, bb0, bw1, bb1, bw2, bb2,
              tw0z, w729, tb0, tw1, tb1, tw2, tb2):
    return pl.pallas_call(
        _tc_body,
        grid=(_NBLK,),
        in_specs=[
            pl.BlockSpec((_BS, 13), lambda i: (i, 0)),
            pl.BlockSpec((_BS, _NSP * _EMB), lambda i: (i, 0)),
            _full((13, 512)), _full((512,)),
            _full((512, 256)), _full((256,)),
            _full((256, 32)), _full((32,)),
            _full((32, 512)), _full((_NF * _NF, 512)), _full((512,)),
            _full((512, 256)), _full((256,)),
            _full((256, 1)), _full((1,)),
        ],
        out_specs=pl.BlockSpec((_BS, 1), lambda i: (i, 0)),
        out_shape=jax.ShapeDtypeStruct((_B, 1), jnp.float32),
    )(dense, embs, bw0, bb0, bw1, bb1, bw2, bb2,
      tw0z, w729, tb0, tw1, tb1, tw2, tb2)


_IU0, _IU1 = np.triu_indices(_NF, k=1)
_W729_ROWS = np.asarray(_IU0 * _NF + _IU1, dtype=np.int32)


def kernel(dense, sparse, tables, bw0, bb0, bw1, bb1, bw2, bb2,
           tw0, tb0, tw1, tb1, tw2, tb2):
    idx = (sparse.astype(jnp.int32)
           + (jnp.arange(_NSP, dtype=jnp.int32) * _VOCAB)[None, :])
    idx = idx.reshape(_NW, _NCH, _CH)
    tables_flat = tables.reshape(_NSP * _VOCAB, _EMB)

    embs_flat = _sc_gather(tables_flat, idx)
    embs = embs_flat.reshape(_B, _NSP * _EMB)

    bf = jnp.bfloat16
    tw0z = tw0[:_EMB].astype(bf)
    w729 = jnp.zeros((_NF * _NF, 512), dtype=bf)
    w729 = w729.at[_W729_ROWS].set(tw0[_EMB:].astype(bf))

    out = _tc_dense(dense, embs,
                    bw0.astype(bf), bb0, bw1.astype(bf), bb1,
                    bw2.astype(bf), bb2,
                    tw0z, w729, tb0,
                    tw1.astype(bf), tb1, tw2.astype(bf), tb2)
    return out.reshape(_B)


# speedup vs baseline: 1.4346x; 1.0131x over previous
import functools

import jax
import jax.numpy as jnp
import numpy as np
from jax import lax
from jax.experimental import pallas as pl
from jax.experimental.pallas import tpu as pltpu
from jax.experimental.pallas import tpu_sc as plsc

_B = 16384
_NSP = 26
_EMB = 32
_VOCAB = 100000
_NF = 27
_ROWS = _B * _NSP

_NC = 2
_NSUB = 16
_NW = _NC * _NSUB
_RPW = _ROWS // _NW
_CH = 128
_NCH = _RPW // _CH

_BS = 1024
_NBLK = _B // _BS


_SCCH = 4
_SCH = _CH * _SCCH
_NSCH = _RPW // _SCH


def _sc_gather_body(tables_h, idx_h, out_h, idx_v, ra, rb, sga, sgb):
    c = lax.axis_index("c")
    s = lax.axis_index("s")
    wid = s * _NC + c
    pltpu.sync_copy(idx_h.at[wid], idx_v)
    base = wid * _RPW

    def fireblock(jj, buf, sem):
        for k in range(_SCCH):
            pltpu.async_copy(tables_h.at[idx_v.at[jj * _SCCH + k]],
                             buf.at[pl.ds(k * _CH, _CH)], sem)

    def drainblock(jj, buf, sem):
        for k in range(_SCCH):
            pltpu.make_async_copy(tables_h.at[idx_v.at[jj * _SCCH + k]],
                                  buf.at[pl.ds(k * _CH, _CH)], sem).wait()

    def wb(jj, buf):
        pltpu.sync_copy(buf, out_h.at[pl.ds(base + jj * _SCH, _SCH)])

    fireblock(0, ra, sga)

    def body(t, carry):
        j0 = 2 * t
        j1 = j0 + 1
        fireblock(j1, rb, sgb)
        drainblock(j0, ra, sga)
        wb(j0, ra)

        @pl.when(t < _NSCH // 2 - 1)
        def _():
            fireblock(j0 + 2, ra, sga)
        drainblock(j1, rb, sgb)
        wb(j1, rb)
        return carry

    lax.fori_loop(0, _NSCH // 2, body, 0)


@jax.jit
def _sc_gather(tables_flat, idx):
    mesh = plsc.VectorSubcoreMesh(
        core_axis_name="c", subcore_axis_name="s",
        num_cores=_NC, num_subcores=_NSUB)
    return pl.kernel(
        _sc_gather_body,
        out_type=jax.ShapeDtypeStruct((_ROWS, _EMB), jnp.float32),
        mesh=mesh,
        scratch_types=[
            pltpu.VMEM((_NCH, _CH), jnp.int32),
            pltpu.VMEM((_SCH, _EMB), jnp.float32),
            pltpu.VMEM((_SCH, _EMB), jnp.float32),
            pltpu.SemaphoreType.DMA,
            pltpu.SemaphoreType.DMA,
        ],
        compiler_params=pltpu.CompilerParams(use_tc_tiling_on_sc=False),
    )(tables_flat, idx)


def _tc_body(dense_r, embs_r, bw0_r, bb0_r, bw1_r, bb1_r, bw2_r, bb2_r,
             tw0z_r, w729_r, tb0_r, tw1_r, tb1_r, tw2_r, tb2_r, out_r):
    f32 = jnp.float32
    bf = jnp.bfloat16
    h = dense_r[...].astype(bf)
    h = jnp.dot(h, bw0_r[...], preferred_element_type=f32) + bb0_r[...][None, :]
    h = jnp.maximum(h, 0).astype(bf)
    h = jnp.dot(h, bw1_r[...], preferred_element_type=f32) + bb1_r[...][None, :]
    h = jnp.maximum(h, 0).astype(bf)
    h = jnp.dot(h, bw2_r[...], preferred_element_type=f32) + bb2_r[...][None, :]
    z0 = jnp.maximum(h, 0)
    z0b = z0.astype(bf)
    embs = embs_r[...].astype(bf)
    t2 = jnp.concatenate([z0b, embs], axis=1)
    t3 = t2.reshape(_BS, _NF, _EMB)
    z = lax.dot_general(t3, t3, (((2,), (2,)), ((0,), (0,))),
                        preferred_element_type=f32)
    zf = z.reshape(_BS, _NF * _NF).astype(bf)
    y = jnp.dot(z0b, tw0z_r[...], preferred_element_type=f32)
    y = y + jnp.dot(zf, w729_r[...], preferred_element_type=f32) + tb0_r[...][None, :]
    h = jnp.maximum(y, 0).astype(bf)
    h = jnp.dot(h, tw1_r[...], preferred_element_type=f32) + tb1_r[...][None, :]
    h = jnp.maximum(h, 0).astype(bf)
    o = jnp.dot(h, tw2_r[...], preferred_element_type=f32) + tb2_r[...][None, :]
    out_r[...] = jax.nn.sigmoid(o)


def _full(shape):
    return pl.BlockSpec(shape, lambda i: tuple(0 for _ in shape))


@jax.jit
def _tc_dense(dense, embs, bw0, bb0, bw1, bb1, bw2, bb2,
              tw0z, w729, tb0, tw1, tb1, tw2, tb2):
    return pl.pallas_call(
        _tc_body,
        grid=(_NBLK,),
        in_specs=[
            pl.BlockSpec((_BS, 13), lambda i: (i, 0)),
            pl.BlockSpec((_BS, _NSP * _EMB), lambda i: (i, 0)),
            _full((13, 512)), _full((512,)),
            _full((512, 256)), _full((256,)),
            _full((256, 32)), _full((32,)),
            _full((32, 512)), _full((_NF * _NF, 512)), _full((512,)),
            _full((512, 256)), _full((256,)),
            _full((256, 1)), _full((1,)),
        ],
        out_specs=pl.BlockSpec((_BS, 1), lambda i: (i, 0)),
        out_shape=jax.ShapeDtypeStruct((_B, 1), jnp.float32),
    )(dense, embs, bw0, bb0, bw1, bb1, bw2, bb2,
      tw0z, w729, tb0, tw1, tb1, tw2, tb2)


_IU0, _IU1 = np.triu_indices(_NF, k=1)
_W729_ROWS = np.asarray(_IU0 * _NF + _IU1, dtype=np.int32)


def kernel(dense, sparse, tables, bw0, bb0, bw1, bb1, bw2, bb2,
           tw0, tb0, tw1, tb1, tw2, tb2):
    idx = (sparse.astype(jnp.int32)
           + (jnp.arange(_NSP, dtype=jnp.int32) * _VOCAB)[None, :])
    idx = idx.reshape(_NW, _NCH, _CH)
    tables_flat = tables.reshape(_NSP * _VOCAB, _EMB)

    embs_flat = _sc_gather(tables_flat, idx)
    embs = embs_flat.reshape(_B, _NSP * _EMB)

    bf = jnp.bfloat16
    tw0z = tw0[:_EMB].astype(bf)
    w729 = jnp.zeros((_NF * _NF, 512), dtype=bf)
    w729 = w729.at[_W729_ROWS].set(tw0[_EMB:].astype(bf))

    out = _tc_dense(dense, embs,
                    bw0.astype(bf), bb0, bw1.astype(bf), bb1,
                    bw2.astype(bf), bb2,
                    tw0z, w729, tb0,
                    tw1.astype(bf), tb1, tw2.astype(bf), tb2)
    return out.reshape(_B)
